# baseline (speedup 1.0000x reference)
import functools

import jax
import jax.numpy as jnp
from jax import lax
from jax.experimental import pallas as pl
from jax.experimental.pallas import tpu as pltpu
from jax.experimental.pallas import tpu_sc as plsc

VERT_NUM = 100000
EMBED_DIM = 400
BATCH = 16384

_INFO = plsc.get_sparse_core_info()
_NC = _INFO.num_cores
_NS = _INFO.num_subcores
_NW = _NC * _NS
_B_PER_W = BATCH // _NW
_CHUNK = 64
_NCHUNK = _B_PER_W // _CHUNK
_TAIL = EMBED_DIM - 384


def _gather_body(x_hbm, table_hbm, out_hbm,
                 idx_v, buf0, buf1, tbuf0, tbuf1, sem0, sem1):
    wid = lax.axis_index("s") * _NC + lax.axis_index("c")
    base = wid * _B_PER_W
    pltpu.sync_copy(x_hbm.at[wid], idx_v)

    bufs = (buf0, buf1)
    tbufs = (tbuf0, tbuf1)
    sems = (sem0, sem1)

    def start(c):
        cps = []
        for off in (0, 128, 256):
            cps.append(
                pltpu.async_copy(
                    table_hbm.at[idx_v.at[c], pl.ds(off, 128)],
                    bufs[c % 2].at[:, pl.ds(off, 128)],
                    sems[c % 2],
                )
            )
        cps.append(
            pltpu.async_copy(table_hbm.at[idx_v.at[c], pl.ds(384, 128)],
                             tbufs[c % 2], sems[c % 2])
        )
        return cps

    copies = [None] * _NCHUNK
    copies[0] = start(0)
    for c in range(_NCHUNK):
        if c + 1 < _NCHUNK:
            copies[c + 1] = start(c + 1)
        for cp in copies[c]:
            cp.wait()
        buf, tbuf = bufs[c % 2], tbufs[c % 2]
        for r in range(_CHUNK):
            buf[r, pl.ds(384, _TAIL)] = tbuf[r, pl.ds(0, _TAIL)]
        pltpu.sync_copy(buf, out_hbm.at[pl.ds(base + c * _CHUNK, _CHUNK)])


@jax.jit
def _gather_sc(x, table):
    table512 = jnp.pad(table, ((0, 0), (0, 512 - EMBED_DIM)))
    kern = functools.partial(
        pl.kernel,
        out_type=jax.ShapeDtypeStruct((BATCH, EMBED_DIM), jnp.float32),
        mesh=plsc.VectorSubcoreMesh(core_axis_name="c", subcore_axis_name="s"),
        scratch_types=[
            pltpu.VMEM((_NCHUNK, _CHUNK), jnp.int32),
            pltpu.VMEM((_CHUNK, EMBED_DIM), jnp.float32),
            pltpu.VMEM((_CHUNK, EMBED_DIM), jnp.float32),
            pltpu.VMEM((_CHUNK, 128), jnp.float32),
            pltpu.VMEM((_CHUNK, 128), jnp.float32),
            pltpu.SemaphoreType.DMA,
            pltpu.SemaphoreType.DMA,
        ],
    )(_gather_body)
    return kern(x.reshape(_NW, _NCHUNK, _CHUNK).astype(jnp.int32), table512)


def kernel(x, table):
    emb = _gather_sc(x, table)
    return emb.reshape(EMBED_DIM, -1)


# speedup vs baseline: 1.9144x; 1.9144x over previous
import functools

import jax
import jax.numpy as jnp
from jax import lax
from jax.experimental import pallas as pl
from jax.experimental.pallas import tpu as pltpu
from jax.experimental.pallas import tpu_sc as plsc

VERT_NUM = 100000
EMBED_DIM = 400
BATCH = 16384

_INFO = plsc.get_sparse_core_info()
_NC = _INFO.num_cores
_NS = _INFO.num_subcores
_NW = _NC * _NS
_RNG = 3200
_WIN = 3200
_CAP = 640
_GROUPS = ((0, 16, 0), (16, 16, 128), (32, 16, 256), (48, 2, 384))
_CLOMAX = VERT_NUM + 1 - 3105 - 0
_OUTROWS = BATCH + _NW


def _body(x_hbm, tt_hbm, tl_hbm, out_hbm,
          xbuf, hbf, hvf, hb2, slab, hit, semo):
    wid = lax.axis_index("s") * _NC + lax.axis_index("c")
    lo = wid * _RNG
    hi = lo + _RNG
    clo = pl.multiple_of(jnp.minimum(lo, _CLOMAX), 128)
    voff = lo - clo
    dump = BATCH + wid
    lane = lax.iota(jnp.int32, 16)
    zeros = jnp.zeros((16,), jnp.int32)
    ones = zeros + 1

    def filter_pass(skip):
        def pre(i, c):
            hbf[pl.ds(i * 16, 16)] = zeros + dump
            hvf[pl.ds(i * 16, 16)] = zeros
            return c
        lax.fori_loop(0, _CAP // 16, pre, jnp.int32(0))

        def half(h, cnt):
            pltpu.sync_copy(x_hbm.at[pl.ds(pl.multiple_of(h * 8192, 8192), 8192)], xbuf)
            def chunk(t, cnt):
                xv = xbuf[pl.ds(t * 16, 16)]
                m = (xv >= lo) & (xv < hi)
                pref = plsc.cumsum(jnp.where(m, ones, zeros))
                ordv = cnt + pref
                keep = m & (ordv > skip) & (ordv <= skip + _CAP)
                pos = jnp.minimum(jnp.maximum(ordv - 1 - skip, 0), _CAP - 1)
                plsc.store_scatter(hbf, [pos], h * 8192 + t * 16 + lane,
                                   mask=keep)
                plsc.store_scatter(hvf, [pos], xv - lo, mask=keep)
                return cnt + jnp.max(pref)
            return lax.fori_loop(0, 512, chunk, cnt)
        cnt = lax.fori_loop(0, 2, half, jnp.int32(0))
        for j in range(5):
            for k in range(8):
                hb2[j, pl.ds(k * 16, 16)] = hbf[pl.ds(j * 128 + k * 16, 16)]
        return cnt

    def scan_group(g0, ns, cbase):
        def slab_step(si, c):
            rows = pl.multiple_of((g0 + si) * 8, 8)
            pltpu.sync_copy(tt_hbm.at[pl.ds(rows, 8), pl.ds(clo, _WIN - 128)],
                            slab.at[:, pl.ds(0, _WIN - 128)])
            pltpu.sync_copy(tl_hbm.at[pl.ds(rows, 8), pl.ds(pl.multiple_of(wid * 128, 128), 128)],
                            slab.at[:, pl.ds(_WIN - 128, 128)])

            def chunk(t, c2):
                rowv = t * 16 + lane
                vv = hvf[pl.ds(t * 16, 16)] + voff
                for f in range(8):
                    vals = plsc.load_gather(slab, [zeros + f, vv])
                    plsc.store_scatter(hit, [rowv, zeros + (si * 8 + f)], vals)
                return c2
            lax.fori_loop(0, _CAP // 16, chunk, jnp.int32(0))
            return c
        lax.fori_loop(0, ns, slab_step, jnp.int32(0))
        cps = []
        for j in range(5):
            cps.append(pltpu.async_copy(
                hit.at[pl.ds(j * 128, 128)],
                out_hbm.at[hb2.at[j], pl.ds(cbase, 128)], semo))
        for cp in cps:
            cp.wait()

    def body_w(carry):
        skip, _ = carry
        total = filter_pass(skip)
        for g0, ns, cbase in _GROUPS:
            scan_group(g0, ns, cbase)
        return (skip + _CAP, total)

    lax.while_loop(lambda c: c[1] > c[0], body_w,
                   (jnp.int32(0), jnp.int32(1)))


@jax.jit
def _gather_sc(x, table):
    tt = table.T
    blocks = []
    for w in range(_NW):
        s0 = min(w * _RNG, _CLOMAX) + (_WIN - 128)
        if s0 + 128 <= VERT_NUM + 1:
            blocks.append(table[s0:s0 + 128])
        else:
            blocks.append(jnp.pad(table[s0:], ((0, s0 + 128 - VERT_NUM - 1),
                                               (0, 0))))
    tl = jnp.concatenate(blocks, axis=0).T
    kern = functools.partial(
        pl.kernel,
        out_type=jax.ShapeDtypeStruct((_OUTROWS, 512), jnp.float32),
        mesh=plsc.VectorSubcoreMesh(core_axis_name="c", subcore_axis_name="s"),
        compiler_params=pltpu.CompilerParams(needs_layout_passes=False),
        scratch_types=[
            pltpu.VMEM((8192,), jnp.int32),
            pltpu.VMEM((_CAP,), jnp.int32),
            pltpu.VMEM((_CAP,), jnp.int32),
            pltpu.VMEM((5, 128), jnp.int32),
            pltpu.VMEM((8, _WIN), jnp.float32),
            pltpu.VMEM((_CAP, 128), jnp.float32),
            pltpu.SemaphoreType.DMA,
        ],
    )(_body)
    res = kern(x.astype(jnp.int32), tt, tl)
    return res[:BATCH, :EMBED_DIM]


def kernel(x, table):
    emb = _gather_sc(x, table)
    return emb.reshape(EMBED_DIM, -1)


# speedup vs baseline: 3.2184x; 1.6812x over previous
import functools

import jax
import jax.numpy as jnp
from jax import lax
from jax.experimental import pallas as pl
from jax.experimental.pallas import tpu as pltpu
from jax.experimental.pallas import tpu_sc as plsc

VERT_NUM = 100000
EMBED_DIM = 400
BATCH = 16384

_INFO = plsc.get_sparse_core_info()
_NC = _INFO.num_cores
_NS = _INFO.num_subcores
_NW = _NC * _NS
_B_PER_W = BATCH // _NW
_CHUNK = 64
_NCHUNK = _B_PER_W // _CHUNK
_TAIL = EMBED_DIM - 384


def _gather_body(x_hbm, table_hbm, tail_hbm, out_hbm,
                 idx_v, buf0, buf1, tbuf0, tbuf1, sem0, sem1):
    wid = lax.axis_index("s") * _NC + lax.axis_index("c")
    base = wid * _B_PER_W
    pltpu.sync_copy(x_hbm.at[wid], idx_v)

    bufs = (buf0, buf1)
    tbufs = (tbuf0, tbuf1)
    sems = (sem0, sem1)

    def start(c):
        cps = []
        for off in (0, 128, 256):
            cps.append(
                pltpu.async_copy(
                    table_hbm.at[idx_v.at[c], pl.ds(off, 128)],
                    bufs[c % 2].at[:, pl.ds(off, 128)],
                    sems[c % 2],
                )
            )
        cps.append(
            pltpu.async_copy(tail_hbm.at[idx_v.at[c]], tbufs[c % 2],
                             sems[c % 2])
        )
        return cps

    copies = [None] * _NCHUNK
    copies[0] = start(0)
    for c in range(_NCHUNK):
        if c + 1 < _NCHUNK:
            copies[c + 1] = start(c + 1)
        for cp in copies[c]:
            cp.wait()
        buf, tbuf = bufs[c % 2], tbufs[c % 2]
        for r in range(_CHUNK):
            buf[r, pl.ds(384, _TAIL)] = tbuf[r, pl.ds(0, _TAIL)]
        pltpu.sync_copy(buf, out_hbm.at[pl.ds(base + c * _CHUNK, _CHUNK)])


@jax.jit
def _gather_sc(x, table):
    tail = jnp.pad(table[:, 384:], ((0, 0), (0, 128 - _TAIL)))
    kern = functools.partial(
        pl.kernel,
        out_type=jax.ShapeDtypeStruct((BATCH, EMBED_DIM), jnp.float32),
        mesh=plsc.VectorSubcoreMesh(core_axis_name="c", subcore_axis_name="s"),
        scratch_types=[
            pltpu.VMEM((_NCHUNK, _CHUNK), jnp.int32),
            pltpu.VMEM((_CHUNK, EMBED_DIM), jnp.float32),
            pltpu.VMEM((_CHUNK, EMBED_DIM), jnp.float32),
            pltpu.VMEM((_CHUNK, 128), jnp.float32),
            pltpu.VMEM((_CHUNK, 128), jnp.float32),
            pltpu.SemaphoreType.DMA,
            pltpu.SemaphoreType.DMA,
        ],
    )(_gather_body)
    return kern(x.reshape(_NW, _NCHUNK, _CHUNK).astype(jnp.int32), table, tail)


def kernel(x, table):
    emb = _gather_sc(x, table)
    return emb.reshape(EMBED_DIM, -1)
